# baseline (speedup 1.0000x reference)
import functools

import jax
import jax.numpy as jnp
from jax import lax
from jax.experimental import pallas as pl
from jax.experimental.pallas import tpu as pltpu
from jax.experimental.pallas import tpu_sc as plsc

_SAMPLE = 8
_CHUNK = 3072
_VLEN = 16


def _pass_kernel(idx_ref, x_ref, xs_ref, s_ref, sq_ref, n_images: int):
    step = pl.program_id(0)
    nsteps = pl.num_programs(0)
    chunk = x_ref.shape[2]
    base = step * chunk

    @pl.when(step == 0)
    def _init():
        s_ref[...] = jnp.zeros_like(s_ref)
        sq_ref[...] = jnp.zeros_like(sq_ref)

    @pl.when(step < nsteps - 1)
    def _full():
        x = x_ref[...]
        s_ref[...] += jnp.sum(x, axis=0)
        sq_ref[...] += jnp.sum(x * x, axis=0)

    for j in range(_SAMPLE):
        idx = idx_ref[j]
        local = idx - base

        @pl.when((idx >= base) & (idx < base + chunk))
        def _copy():
            lane = jax.lax.broadcasted_iota(jnp.int32, x_ref.shape, 2)
            img = jnp.sum(jnp.where(lane == local, x_ref[...], 0.0), axis=2,
                          keepdims=True)
            out_lane = jax.lax.broadcasted_iota(jnp.int32, xs_ref.shape, 2)
            bcast = jnp.broadcast_to(img, xs_ref.shape)
            xs_ref[...] = jnp.where(out_lane == j, bcast, xs_ref[...])

    @pl.when(step == nsteps - 1)
    def _last():
        valid = n_images - base
        lane = jax.lax.broadcasted_iota(jnp.int32, x_ref.shape, 2)
        x = jnp.where(lane < valid, x_ref[...], 0.0)
        s = s_ref[...] + jnp.sum(x, axis=0)
        sq = sq_ref[...] + jnp.sum(x * x, axis=0)

        total = jnp.float32(x_ref.shape[0] * x_ref.shape[1]) * n_images
        mean = jnp.sum(s) / total
        var = jnp.sum(sq) / total - mean * mean
        inv_std = jax.lax.rsqrt(var)
        xs_ref[...] = (xs_ref[...] - mean) * inv_std

def _ysum_kernel(idxmat_hbm, y_hbm, out_hbm, idxm_v, rows_v, acc_v, sem):
    wid = lax.axis_index("s") * 2 + lax.axis_index("c")

    @pl.when(wid == 0)
    def _go():
        pltpu.sync_copy(idxmat_hbm, idxm_v)
        copies = [
            pltpu.async_copy(y_hbm.at[idxm_v.at[k]], rows_v.at[k], sem)
            for k in range(_SAMPLE)
        ]
        for c in copies:
            c.wait()
        v = rows_v[0, :]
        for k in range(1, _SAMPLE):
            v = v + rows_v[k, :]
        acc_v[...] = v
        pltpu.sync_copy(acc_v, out_hbm)


def kernel(x_train, y_train, indices):
    n, h, w = x_train.shape
    xt = x_train.transpose(1, 2, 0)
    grid = (n + _CHUNK - 1) // _CHUNK

    lanes = jnp.arange(_VLEN)[None, :]
    rolls = jnp.arange(_SAMPLE)[:, None]
    idxmat = indices[(lanes + rolls) % _SAMPLE]
    mesh = plsc.VectorSubcoreMesh(core_axis_name="c", subcore_axis_name="s")
    ysum16 = pl.kernel(
        _ysum_kernel,
        mesh=mesh,
        out_type=jax.ShapeDtypeStruct((_VLEN,), jnp.int32),
        scratch_types=[
            pltpu.VMEM((_SAMPLE, _VLEN), jnp.int32),
            pltpu.VMEM((_SAMPLE, _VLEN), jnp.int32),
            pltpu.VMEM((_VLEN,), jnp.int32),
            pltpu.SemaphoreType.DMA,
        ],
    )(idxmat, y_train)

    xs = pl.pallas_call(
        functools.partial(_pass_kernel, n_images=n),
        grid=(grid,),
        in_specs=[
            pl.BlockSpec(memory_space=pltpu.SMEM),
            pl.BlockSpec((h, w, _CHUNK), lambda i: (0, 0, i)),
        ],
        out_specs=pl.BlockSpec((h, w, _SAMPLE), lambda i: (0, 0, 0)),
        out_shape=jax.ShapeDtypeStruct((h, w, _SAMPLE), jnp.float32),
        scratch_shapes=[
            pltpu.VMEM((w, _CHUNK), jnp.float32),
            pltpu.VMEM((w, _CHUNK), jnp.float32),
        ],
    )(indices, xt)
    return xs.transpose(2, 0, 1), ysum16[0]


# speedup vs baseline: 1.0011x; 1.0011x over previous
import functools

import jax
import jax.numpy as jnp
from jax import lax
from jax.experimental import pallas as pl
from jax.experimental.pallas import tpu as pltpu
from jax.experimental.pallas import tpu_sc as plsc

_SAMPLE = 8
_CHUNK = 3072
_VLEN = 16


def _pass_kernel(idx_ref, x_ref, xs_ref, s_ref, sq_ref, n_images: int):
    step = pl.program_id(0)
    nsteps = pl.num_programs(0)
    chunk = x_ref.shape[2]
    base = step * chunk

    @pl.when(step == 0)
    def _init():
        s_ref[...] = jnp.zeros_like(s_ref)
        sq_ref[...] = jnp.zeros_like(sq_ref)

    @pl.when(step < nsteps - 1)
    def _full():
        x = x_ref[...]
        s_ref[...] += jnp.sum(x, axis=0)
        sq_ref[...] += jnp.sum(x * x, axis=0)

    for j in range(_SAMPLE):
        idx = idx_ref[j]
        local = idx - base

        @pl.when((idx >= base) & (idx < base + chunk))
        def _copy():
            lane = jax.lax.broadcasted_iota(jnp.int32, x_ref.shape, 2)
            img = jnp.sum(jnp.where(lane == local, x_ref[...], 0.0), axis=2,
                          keepdims=True)
            out_lane = jax.lax.broadcasted_iota(jnp.int32, xs_ref.shape, 2)
            bcast = jnp.broadcast_to(img, xs_ref.shape)
            xs_ref[...] = jnp.where(out_lane == j, bcast, xs_ref[...])

    @pl.when(step == nsteps - 1)
    def _last():
        valid = n_images - base
        lane = jax.lax.broadcasted_iota(jnp.int32, x_ref.shape, 2)
        x = jnp.where(lane < valid, x_ref[...], 0.0)
        s = s_ref[...] + jnp.sum(x, axis=0)
        sq = sq_ref[...] + jnp.sum(x * x, axis=0)

        total = jnp.float32(x_ref.shape[0] * x_ref.shape[1]) * n_images
        mean = jnp.sum(s) / total
        var = jnp.sum(sq) / total - mean * mean
        inv_std = jax.lax.rsqrt(var)
        xs_ref[...] = (xs_ref[...] - mean) * inv_std

def _ysum_kernel(idxmat_hbm, y_hbm, out_hbm, idxm_v, rows_v, acc_v, sem):
    wid = lax.axis_index("s") * 2 + lax.axis_index("c")

    @pl.when(wid == 0)
    def _go():
        pltpu.sync_copy(idxmat_hbm, idxm_v)
        copies = [
            pltpu.async_copy(y_hbm.at[idxm_v.at[k]], rows_v.at[k], sem)
            for k in range(_SAMPLE)
        ]
        for c in copies:
            c.wait()
        v = rows_v[0, :]
        for k in range(1, _SAMPLE):
            v = v + rows_v[k, :]
        acc_v[...] = v
        pltpu.sync_copy(acc_v, out_hbm)


def kernel(x_train, y_train, indices):
    n, h, w = x_train.shape
    xt = x_train.transpose(1, 2, 0)
    grid = (n + _CHUNK - 1) // _CHUNK

    xs = pl.pallas_call(
        functools.partial(_pass_kernel, n_images=n),
        grid=(grid,),
        in_specs=[
            pl.BlockSpec(memory_space=pltpu.SMEM),
            pl.BlockSpec((h, w, _CHUNK), lambda i: (0, 0, i)),
        ],
        out_specs=pl.BlockSpec((h, w, _SAMPLE), lambda i: (0, 0, 0)),
        out_shape=jax.ShapeDtypeStruct((h, w, _SAMPLE), jnp.float32),
        scratch_shapes=[
            pltpu.VMEM((w, _CHUNK), jnp.float32),
            pltpu.VMEM((w, _CHUNK), jnp.float32),
        ],
    )(indices, xt)

    lanes = jnp.arange(_VLEN)[None, :]
    rolls = jnp.arange(_SAMPLE)[:, None]
    idxmat = indices[(lanes + rolls) % _SAMPLE]
    mesh = plsc.VectorSubcoreMesh(core_axis_name="c", subcore_axis_name="s")
    ysum16 = pl.kernel(
        _ysum_kernel,
        mesh=mesh,
        out_type=jax.ShapeDtypeStruct((_VLEN,), jnp.int32),
        scratch_types=[
            pltpu.VMEM((_SAMPLE, _VLEN), jnp.int32),
            pltpu.VMEM((_SAMPLE, _VLEN), jnp.int32),
            pltpu.VMEM((_VLEN,), jnp.int32),
            pltpu.SemaphoreType.DMA,
        ],
    )(idxmat, y_train)
    return xs.transpose(2, 0, 1), ysum16[0]


# speedup vs baseline: 1.1575x; 1.1562x over previous
import jax
import jax.numpy as jnp
from jax.experimental import pallas as pl
from jax.experimental.pallas import tpu as pltpu

_SAMPLE = 8
_CHUNK = 3072


def _pass_kernel(idx_ref, x_ref, y_ref, xs_ref, ysum_ref, s_ref, sq_ref,
                 n_images: int):
    step = pl.program_id(0)
    nsteps = pl.num_programs(0)
    chunk = x_ref.shape[2]
    base = step * chunk

    @pl.when(step == 0)
    def _init():
        s_ref[...] = jnp.zeros_like(s_ref)
        sq_ref[...] = jnp.zeros_like(sq_ref)

    @pl.when(step < nsteps - 1)
    def _full():
        x = x_ref[...]
        s_ref[...] += jnp.sum(x, axis=0)
        sq_ref[...] += jnp.sum(x * x, axis=0)

    for j in range(_SAMPLE):
        idx = idx_ref[j]
        local = idx - base

        @pl.when((idx >= base) & (idx < base + chunk))
        def _copy():
            lane = jax.lax.broadcasted_iota(jnp.int32, x_ref.shape, 2)
            img = jnp.sum(jnp.where(lane == local, x_ref[...], 0.0), axis=2,
                          keepdims=True)
            out_lane = jax.lax.broadcasted_iota(jnp.int32, xs_ref.shape, 2)
            bcast = jnp.broadcast_to(img, xs_ref.shape)
            xs_ref[...] = jnp.where(out_lane == j, bcast, xs_ref[...])

    @pl.when(step == nsteps - 1)
    def _last():
        valid = n_images - base
        lane = jax.lax.broadcasted_iota(jnp.int32, x_ref.shape, 2)
        x = jnp.where(lane < valid, x_ref[...], 0.0)
        s = s_ref[...] + jnp.sum(x, axis=0)
        sq = sq_ref[...] + jnp.sum(x * x, axis=0)

        total = jnp.float32(x_ref.shape[0] * x_ref.shape[1]) * n_images
        mean = jnp.sum(s) / total
        var = jnp.sum(sq) / total - mean * mean
        inv_std = jax.lax.rsqrt(var)
        xs_ref[...] = (xs_ref[...] - mean) * inv_std

        yv = y_ref[...]
        ylane = jax.lax.broadcasted_iota(jnp.int32, yv.shape, 1)
        hits = jnp.zeros_like(yv)
        for j in range(_SAMPLE):
            hits += jnp.where(ylane == idx_ref[j], 1, 0)
        ysum_ref[0, 0] = jnp.sum(yv * hits)


def kernel(x_train, y_train, indices):
    n, h, w = x_train.shape
    xt = x_train.transpose(1, 2, 0)
    y2 = y_train.reshape(1, n)
    grid = (n + _CHUNK - 1) // _CHUNK

    import functools

    xs, ysum = pl.pallas_call(
        functools.partial(_pass_kernel, n_images=n),
        grid=(grid,),
        in_specs=[
            pl.BlockSpec(memory_space=pltpu.SMEM),
            pl.BlockSpec((h, w, _CHUNK), lambda i: (0, 0, i)),
            pl.BlockSpec((1, n), lambda i: (0, 0)),
        ],
        out_specs=[
            pl.BlockSpec((h, w, _SAMPLE), lambda i: (0, 0, 0)),
            pl.BlockSpec(memory_space=pltpu.SMEM),
        ],
        out_shape=[
            jax.ShapeDtypeStruct((h, w, _SAMPLE), jnp.float32),
            jax.ShapeDtypeStruct((1, 1), y_train.dtype),
        ],
        scratch_shapes=[
            pltpu.VMEM((w, _CHUNK), jnp.float32),
            pltpu.VMEM((w, _CHUNK), jnp.float32),
        ],
    )(indices, xt, y2)
    return xs.transpose(2, 0, 1), ysum[0, 0]
